# baseline (speedup 1.0000x reference)
import jax
import jax.numpy as jnp
from jax.experimental import pallas as pl
from jax.experimental.pallas import tpu as pltpu

CHUNK = 10_000
SLOTS = 8
U_CHUNKS = 1_000_000 // CHUNK
I_CHUNKS = 100_000 // CHUNK
N = U_CHUNKS + I_CHUNKS


def _copy_body(u_in, i_in, u_out, i_out, bufs, in_sems, out_sems):
    chunks = [(0, c * CHUNK) for c in range(U_CHUNKS)] + [
        (1, c * CHUNK) for c in range(I_CHUNKS)
    ]
    srcs = [u_in, i_in]
    dsts = [u_out, i_out]

    def in_copy(c, slot):
        t, b = chunks[c]
        return pltpu.make_async_copy(
            srcs[t].at[pl.ds(b, CHUNK)], bufs.at[slot], in_sems.at[slot])

    def out_copy(c, slot):
        t, b = chunks[c]
        return pltpu.make_async_copy(
            bufs.at[slot], dsts[t].at[pl.ds(b, CHUNK)], out_sems.at[slot])

    for s in range(SLOTS):
        in_copy(s, s).start()
    for c in range(N):
        slot = c % SLOTS
        in_copy(c, slot).wait()
        out_copy(c, slot).start()
        d = c - SLOTS // 2
        if d >= 0 and d + SLOTS < N:
            slot2 = d % SLOTS
            out_copy(d, slot2).wait()
            in_copy(d + SLOTS, slot2).start()
    for d in range(max(0, N - SLOTS), N):
        out_copy(d, d % SLOTS).wait()


def kernel(user_emb, item_emb):
    return pl.pallas_call(
        _copy_body,
        in_specs=[
            pl.BlockSpec(memory_space=pltpu.MemorySpace.HBM),
            pl.BlockSpec(memory_space=pltpu.MemorySpace.HBM),
        ],
        out_specs=[
            pl.BlockSpec(memory_space=pltpu.MemorySpace.HBM),
            pl.BlockSpec(memory_space=pltpu.MemorySpace.HBM),
        ],
        out_shape=[
            jax.ShapeDtypeStruct(user_emb.shape, user_emb.dtype),
            jax.ShapeDtypeStruct(item_emb.shape, item_emb.dtype),
        ],
        scratch_shapes=[
            pltpu.VMEM((SLOTS, CHUNK, 16), jnp.float32),
            pltpu.SemaphoreType.DMA((SLOTS,)),
            pltpu.SemaphoreType.DMA((SLOTS,)),
        ],
    )(user_emb, item_emb)


# speedup vs baseline: 1.0014x; 1.0014x over previous
import jax
import jax.numpy as jnp
from jax.experimental import pallas as pl
from jax.experimental.pallas import tpu as pltpu

CHUNK = 25_000
SLOTS = 4
U_CHUNKS = 1_000_000 // CHUNK
I_CHUNKS = 100_000 // CHUNK
N = U_CHUNKS + I_CHUNKS


def _copy_body(u_in, i_in, u_out, i_out, bufs, in_sems, out_sems):
    chunks = [(0, c * CHUNK) for c in range(U_CHUNKS)] + [
        (1, c * CHUNK) for c in range(I_CHUNKS)
    ]
    srcs = [u_in, i_in]
    dsts = [u_out, i_out]

    def in_copy(c, slot):
        t, b = chunks[c]
        return pltpu.make_async_copy(
            srcs[t].at[pl.ds(b, CHUNK)], bufs.at[slot], in_sems.at[slot])

    def out_copy(c, slot):
        t, b = chunks[c]
        return pltpu.make_async_copy(
            bufs.at[slot], dsts[t].at[pl.ds(b, CHUNK)], out_sems.at[slot])

    for s in range(SLOTS):
        in_copy(s, s).start()
    for c in range(N):
        slot = c % SLOTS
        in_copy(c, slot).wait()
        out_copy(c, slot).start()
        d = c - SLOTS // 2
        if d >= 0 and d + SLOTS < N:
            slot2 = d % SLOTS
            out_copy(d, slot2).wait()
            in_copy(d + SLOTS, slot2).start()
    for d in range(max(0, N - SLOTS), N):
        out_copy(d, d % SLOTS).wait()


def kernel(user_emb, item_emb):
    return pl.pallas_call(
        _copy_body,
        in_specs=[
            pl.BlockSpec(memory_space=pltpu.MemorySpace.HBM),
            pl.BlockSpec(memory_space=pltpu.MemorySpace.HBM),
        ],
        out_specs=[
            pl.BlockSpec(memory_space=pltpu.MemorySpace.HBM),
            pl.BlockSpec(memory_space=pltpu.MemorySpace.HBM),
        ],
        out_shape=[
            jax.ShapeDtypeStruct(user_emb.shape, user_emb.dtype),
            jax.ShapeDtypeStruct(item_emb.shape, item_emb.dtype),
        ],
        scratch_shapes=[
            pltpu.VMEM((SLOTS, CHUNK, 16), jnp.float32),
            pltpu.SemaphoreType.DMA((SLOTS,)),
            pltpu.SemaphoreType.DMA((SLOTS,)),
        ],
    )(user_emb, item_emb)
